# Initial kernel scaffold; baseline (speedup 1.0000x reference)
import jax
import jax.numpy as jnp
from jax.experimental import pallas as pl


def kernel(x, knn0, w0p, b0p, w0n, b0n, knn1, w1p, b1p, w1n, b1n, knn2, w2p, b2p, w2n, b2n, fcw_p, fcb_p, fcw_n, fcb_n, fc3w, fc3b):
    raise NotImplementedError("write your pallas kernel here")


# speedup vs baseline: 1.3135x; 1.3135x over previous
import jax
import jax.numpy as jnp
from jax import lax
from jax.experimental import pallas as pl
from jax.experimental.pallas import tpu as pltpu
from jax.experimental.pallas import tpu_sc as plsc

_B = 1024
_IN_DIM = 10000
_K = 16
_LANES = 16
_NC, _NS = 2, 16
_NW = _NC * _NS
_ROWS_PER_W = _B // _NW
_NCHUNK = 10


def _lcn_layer(h, packed, d_pad, F, R):
    prev_w = h.shape[1]
    R = int(R)
    G = _ROWS_PER_W // R
    P = _NCHUNK // 2

    def compute_chunk(tab_v, h_v, out_v):
        def j_body(j, _):
            j0 = j * _LANES
            bv = plsc.bitcast(tab_v[2 * _K, pl.ds(j0, _LANES)], jnp.float32)
            accs = [bv for _ in range(R)]
            for k in range(_K):
                idx = tab_v[k, pl.ds(j0, _LANES)]
                wv = plsc.bitcast(tab_v[_K + k, pl.ds(j0, _LANES)], jnp.float32)
                for r in range(R):
                    gv = plsc.load_gather(h_v.at[r], [idx])
                    accs[r] = accs[r] + wv * gv
            for r in range(R):
                out_v[r, pl.ds(j0, _LANES)] = jnp.maximum(accs[r], 0.0)
            return 0

        lax.fori_loop(0, F // _LANES, j_body, 0)

    def body(h_hbm, tab_hbm, out_hbm,
             h_v, tab_a, tab_b, out_a, out_b,
             sem_a, sem_b, sem_oa, sem_ob):
        wid = lax.axis_index("s") * _NC + lax.axis_index("c")

        def tab_dma(c, buf, sem):
            return pltpu.make_async_copy(tab_hbm.at[c], buf, sem)

        def out_dma(base, c, buf, sem):
            return pltpu.make_async_copy(
                buf, out_hbm.at[pl.ds(base, R), pl.ds(c * F, F)], sem)

        for g in range(G):
            base = wid * _ROWS_PER_W + g * R
            pltpu.sync_copy(h_hbm.at[pl.ds(base, R), :], h_v)
            tab_dma(0, tab_a, sem_a).start()

            def pair_body(p, _):
                c0 = 2 * p
                tab_dma(c0 + 1, tab_b, sem_b).start()
                tab_dma(c0, tab_a, sem_a).wait()

                @pl.when(p > 0)
                def _():
                    out_dma(base, c0 - 2, out_a, sem_oa).wait()

                compute_chunk(tab_a, h_v, out_a)
                out_dma(base, c0, out_a, sem_oa).start()

                @pl.when(p < P - 1)
                def _():
                    tab_dma(c0 + 2, tab_a, sem_a).start()

                tab_dma(c0 + 1, tab_b, sem_b).wait()

                @pl.when(p > 0)
                def _():
                    out_dma(base, c0 - 1, out_b, sem_ob).wait()

                compute_chunk(tab_b, h_v, out_b)
                out_dma(base, c0 + 1, out_b, sem_ob).start()
                return 0

            lax.fori_loop(0, P, pair_body, 0)
            out_dma(base, _NCHUNK - 2, out_a, sem_oa).wait()
            out_dma(base, _NCHUNK - 1, out_b, sem_ob).wait()

    mesh = plsc.VectorSubcoreMesh(core_axis_name="c", subcore_axis_name="s")
    fn = pl.kernel(
        body,
        out_type=jax.ShapeDtypeStruct((_B, d_pad), jnp.float32),
        mesh=mesh,
        compiler_params=pltpu.CompilerParams(use_tc_tiling_on_sc=False,
                                             needs_layout_passes=False),
        scratch_types=[
            pltpu.VMEM((R, prev_w), jnp.float32),
            pltpu.VMEM((2 * _K + 1, F), jnp.int32),
            pltpu.VMEM((2 * _K + 1, F), jnp.int32),
            pltpu.VMEM((R, F), jnp.float32),
            pltpu.VMEM((R, F), jnp.float32),
            pltpu.SemaphoreType.DMA,
            pltpu.SemaphoreType.DMA,
            pltpu.SemaphoreType.DMA,
            pltpu.SemaphoreType.DMA,
        ],
    )
    return fn(h, packed)


def _prep_tables(knn, w, b, d_pad):
    d = knn.shape[0]
    F = d_pad // _NCHUNK
    knnT = jnp.zeros((_K, d_pad), jnp.int32).at[:, :d].set(knn.T)
    wT = jnp.zeros((_K, d_pad), jnp.float32).at[:, :d].set(w.T)
    bp = jnp.zeros((1, d_pad), jnp.float32).at[0, :d].set(b.reshape(-1))
    wi = lax.bitcast_convert_type(wT, jnp.int32)
    bi = lax.bitcast_convert_type(bp, jnp.int32)
    packed = jnp.concatenate([knnT, wi, bi], axis=0)
    packed = packed.reshape(2 * _K + 1, _NCHUNK, F).transpose(1, 0, 2)
    return packed, F


_D_PADS = (5120, 2560, 1280)
_RS = (8, 16, 32)


def kernel(x, knn0, w0p, b0p, w0n, b0n, knn1, w1p, b1p, w1n, b1n,
           knn2, w2p, b2p, w2n, b2n, fcw_p, fcb_p, fcw_n, fcb_n,
           fc3w, fc3b):
    knns = (knn0, knn1, knn2)

    def run_net(h, ws, bs):
        for i in range(3):
            packed, F = _prep_tables(knns[i], ws[i], bs[i], _D_PADS[i])
            h = _lcn_layer(h, packed, _D_PADS[i], F, _RS[i])
        return h[:, :1250]

    hp = run_net(x[:, :_IN_DIM], (w0p, w1p, w2p), (b0p, b1p, b2p))
    hn = run_net(x[:, _IN_DIM:], (w0n, w1n, w2n), (b0n, b1n, b2n))
    xp = hp @ fcw_p.T + fcb_p
    xn = hn @ fcw_n.T + fcb_n
    h = jnp.maximum(jnp.concatenate([xp, xn], axis=1), 0.0)
    return h @ fc3w.T + fc3b


# speedup vs baseline: 1.6697x; 1.2712x over previous
import jax
import jax.numpy as jnp
from jax import lax
from jax.experimental import pallas as pl
from jax.experimental.pallas import tpu as pltpu
from jax.experimental.pallas import tpu_sc as plsc

_B = 1024
_IN_DIM = 10000
_K = 16
_LANES = 16
_NC, _NS = 2, 16
_NW = _NC * _NS
_ROWS_PER_W = _B // _NW
_NCHUNK = 10


def _lcn_layer(h, packed, d_pad, F, R):
    prev_w = h.shape[1]
    R = int(R)
    G = _ROWS_PER_W // R
    P = _NCHUNK // 2

    def compute_chunk(tab_v, h_v, out_v):
        def j_body(j, _):
            j0 = j * _LANES
            bv = plsc.bitcast(tab_v[2 * _K, pl.ds(j0, _LANES)], jnp.float32)
            accs = [bv for _ in range(R)]
            for k in range(_K):
                idx = tab_v[k, pl.ds(j0, _LANES)]
                wv = plsc.bitcast(tab_v[_K + k, pl.ds(j0, _LANES)], jnp.float32)
                for r in range(R):
                    gv = plsc.load_gather(h_v.at[r], [idx])
                    accs[r] = accs[r] + wv * gv
            for r in range(R):
                out_v[r, pl.ds(j0, _LANES)] = jnp.maximum(accs[r], 0.0)
            return 0

        lax.fori_loop(0, F // _LANES, j_body, 0)

    def body(h_hbm, tab_hbm, out_hbm,
             h_v, tab_a, tab_b, out_a, out_b,
             sem_a, sem_b, sem_oa, sem_ob):
        wid = lax.axis_index("s") * _NC + lax.axis_index("c")

        def tab_dma(c, buf, sem):
            return pltpu.make_async_copy(tab_hbm.at[c], buf, sem)

        def out_dma(base, c, buf, sem):
            return pltpu.make_async_copy(
                buf, out_hbm.at[pl.ds(base, R), pl.ds(c * F, F)], sem)

        for g in range(G):
            base = wid * _ROWS_PER_W + g * R
            pltpu.sync_copy(h_hbm.at[pl.ds(base, R), :], h_v)
            tab_dma(0, tab_a, sem_a).start()

            def pair_body(p, _):
                c0 = 2 * p
                tab_dma(c0 + 1, tab_b, sem_b).start()
                tab_dma(c0, tab_a, sem_a).wait()

                @pl.when(p > 0)
                def _():
                    out_dma(base, c0 - 2, out_a, sem_oa).wait()

                compute_chunk(tab_a, h_v, out_a)
                out_dma(base, c0, out_a, sem_oa).start()

                @pl.when(p < P - 1)
                def _():
                    tab_dma(c0 + 2, tab_a, sem_a).start()

                tab_dma(c0 + 1, tab_b, sem_b).wait()

                @pl.when(p > 0)
                def _():
                    out_dma(base, c0 - 1, out_b, sem_ob).wait()

                compute_chunk(tab_b, h_v, out_b)
                out_dma(base, c0 + 1, out_b, sem_ob).start()
                return 0

            lax.fori_loop(0, P, pair_body, 0)
            out_dma(base, _NCHUNK - 2, out_a, sem_oa).wait()
            out_dma(base, _NCHUNK - 1, out_b, sem_ob).wait()

    mesh = plsc.VectorSubcoreMesh(core_axis_name="c", subcore_axis_name="s")
    fn = pl.kernel(
        body,
        out_type=jax.ShapeDtypeStruct((_B, d_pad), jnp.float32),
        mesh=mesh,
        compiler_params=pltpu.CompilerParams(use_tc_tiling_on_sc=False,
                                             needs_layout_passes=False),
        scratch_types=[
            pltpu.VMEM((R, prev_w), jnp.float32),
            pltpu.VMEM((2 * _K + 1, F), jnp.int32),
            pltpu.VMEM((2 * _K + 1, F), jnp.int32),
            pltpu.VMEM((R, F), jnp.float32),
            pltpu.VMEM((R, F), jnp.float32),
            pltpu.SemaphoreType.DMA,
            pltpu.SemaphoreType.DMA,
            pltpu.SemaphoreType.DMA,
            pltpu.SemaphoreType.DMA,
        ],
    )
    return fn(h, packed)


def _prep_tables(knn, w, b, d_pad):
    d = knn.shape[0]
    F = d_pad // _NCHUNK
    knnT = jnp.zeros((_K, d_pad), jnp.int32).at[:, :d].set(knn.T)
    wT = jnp.zeros((_K, d_pad), jnp.float32).at[:, :d].set(w.T)
    bp = jnp.zeros((1, d_pad), jnp.float32).at[0, :d].set(b.reshape(-1))
    wi = lax.bitcast_convert_type(wT, jnp.int32)
    bi = lax.bitcast_convert_type(bp, jnp.int32)
    packed = jnp.concatenate([knnT, wi, bi], axis=0)
    packed = packed.reshape(2 * _K + 1, _NCHUNK, F).transpose(1, 0, 2)
    return packed, F


_D_PADS = (5120, 2560, 1280)
_RS = (8, 8, 8)


def kernel(x, knn0, w0p, b0p, w0n, b0n, knn1, w1p, b1p, w1n, b1n,
           knn2, w2p, b2p, w2n, b2n, fcw_p, fcb_p, fcw_n, fcb_n,
           fc3w, fc3b):
    knns = (knn0, knn1, knn2)

    def run_net(h, ws, bs):
        for i in range(3):
            packed, F = _prep_tables(knns[i], ws[i], bs[i], _D_PADS[i])
            h = _lcn_layer(h, packed, _D_PADS[i], F, _RS[i])
        return h[:, :1250]

    hp = run_net(x[:, :_IN_DIM], (w0p, w1p, w2p), (b0p, b1p, b2p))
    hn = run_net(x[:, _IN_DIM:], (w0n, w1n, w2n), (b0n, b1n, b2n))
    xp = hp @ fcw_p.T + fcb_p
    xn = hn @ fcw_n.T + fcb_n
    h = jnp.maximum(jnp.concatenate([xp, xn], axis=1), 0.0)
    return h @ fc3w.T + fc3b
